# baseline (speedup 1.0000x reference)
import functools

import jax
import jax.numpy as jnp
from jax import lax
from jax.experimental import pallas as pl
from jax.experimental.pallas import tpu as pltpu, tpu_sc as plsc

GROUP = 256
NBUF = 2


@functools.lru_cache(maxsize=None)
def _build(A: int, Bdim: int, D: int):
    info = plsc.get_sparse_core_info()
    NC, NS = info.num_cores, info.num_subcores
    NW = NC * NS
    B = A * Bdim
    n_slab = D // 8
    n_chunk = A // 128
    kp = GROUP // 128
    groups_per_b = A // GROUP
    n_groups = Bdim * groups_per_b
    assert n_groups % NW == 0
    g_per_w = n_groups // NW
    n_t = n_slab * kp * 8

    mesh = plsc.VectorSubcoreMesh(core_axis_name="c", subcore_axis_name="s")

    @functools.partial(
        pl.kernel,
        out_type=jax.ShapeDtypeStruct((B * D,), jnp.float32),
        mesh=mesh,
        scratch_types=[
            pltpu.VMEM((NBUF, GROUP), jnp.int32),
            pltpu.VMEM((NBUF, GROUP, D), jnp.float32),
            pltpu.VMEM((NBUF, GROUP * D), jnp.float32),
        ] + [pltpu.SemaphoreType.DMA] * (2 * NBUF),
        compiler_params=pltpu.CompilerParams(
            use_tc_tiling_on_sc=False, needs_layout_passes=False
        ),
    )
    def emb(table_hbm, idx_hbm, out_hbm, idx_v, rows_v, tbuf_v, *sems):
        gsem, wsem = sems[:NBUF], sems[NBUF:]
        wid = lax.axis_index("s") * NC + lax.axis_index("c")
        g0 = wid * g_per_w
        iota = lax.iota(jnp.int32, 16)

        def fetch(g, bf):
            pltpu.sync_copy(idx_hbm.at[pl.ds(g * GROUP, GROUP)], idx_v.at[bf])
            pltpu.async_copy(
                table_hbm.at[idx_v.at[bf]], rows_v.at[bf], gsem[bf]
            )

        def gather_wait(bf):
            pltpu.make_async_copy(
                table_hbm.at[idx_v.at[bf]], rows_v.at[bf], gsem[bf]
            ).wait()

        def write_descs(g, bf):
            b = g // groups_per_b
            k0 = (g % groups_per_b) * kp
            descs = []
            for i in range(n_slab):
                off = ((b * n_slab + i) * n_chunk + k0) * 1024
                descs.append(
                    pltpu.make_async_copy(
                        tbuf_v.at[bf, pl.ds(i * kp * 1024, kp * 1024)],
                        out_hbm.at[pl.ds(off, kp * 1024)],
                        wsem[bf],
                    )
                )
            return descs

        def transpose(bf):
            @pl.loop(0, n_t, unroll=8)
            def tstep(t):
                i = t // (kp * 8)
                rem = t - i * (kp * 8)
                k = rem // 8
                s = rem - k * 8
                col = i * 8 + s
                base = k * 128 + iota
                c_idx = jnp.broadcast_to(col, (16,)).astype(jnp.int32)
                for lb in range(8):
                    v = plsc.load_gather(
                        rows_v.at[bf], [base + lb * 16, c_idx]
                    )
                    tbuf_v[bf, pl.ds(t * 128 + lb * 16, 16)] = v

        for bf in range(NBUF):
            fetch(g0 + bf, bf)

        for bf in range(NBUF):
            g = g0 + bf
            gather_wait(bf)
            transpose(bf)
            for d in write_descs(g, bf):
                d.start()
            if NBUF < g_per_w:
                fetch(g + NBUF, bf)

        @pl.loop(NBUF, g_per_w)
        def grp(gr):
            g = g0 + gr
            bf = lax.rem(gr, NBUF)

            def body(bf_static):
                gather_wait(bf_static)
                for d in write_descs(g - NBUF, bf_static):
                    d.wait()
                transpose(bf_static)
                for d in write_descs(g, bf_static):
                    d.start()

                @pl.when(gr + NBUF < g_per_w)
                def _():
                    fetch(g + NBUF, bf_static)

            for bf_static in range(NBUF):

                @pl.when(bf == bf_static)
                def _():
                    body(bf_static)

        for bf in range(NBUF):
            g = g0 + g_per_w - NBUF + bf
            for d in write_descs(g, bf):
                d.wait()

    return emb


def kernel(x, table):
    A, Bdim = x.shape
    D = table.shape[1]
    xf = jnp.swapaxes(x, 0, 1).reshape(-1).astype(jnp.int32)
    tbl_lin = lax.optimization_barrier(
        table.reshape(table.shape[0] // 2, 2 * D)
    )
    tbl = tbl_lin.reshape(table.shape)
    out_flat = _build(A, Bdim, D)(tbl, xf)
    out5 = out_flat.reshape(Bdim, D // 8, A // 128, 8, 128)
    return out5.transpose(2, 4, 0, 1, 3).reshape(A, Bdim, D)


# speedup vs baseline: 1.3889x; 1.3889x over previous
import functools

import jax
import jax.numpy as jnp
from jax import lax
from jax.experimental import pallas as pl
from jax.experimental.pallas import tpu as pltpu, tpu_sc as plsc

GROUP = 256
NBUF = 2


@functools.lru_cache(maxsize=None)
def _build(A: int, Bdim: int, D: int):
    info = plsc.get_sparse_core_info()
    NC, NS = info.num_cores, info.num_subcores
    NW = NC * NS
    B = A * Bdim
    n_slab = D // 8
    n_chunk = A // 128
    kp = GROUP // 128
    groups_per_b = A // GROUP
    n_groups = Bdim * groups_per_b
    assert n_groups % NW == 0
    g_per_w = n_groups // NW
    n_t = n_slab * kp * 8

    mesh = plsc.VectorSubcoreMesh(core_axis_name="c", subcore_axis_name="s")

    @functools.partial(
        pl.kernel,
        out_type=jax.ShapeDtypeStruct((B * D,), jnp.float32),
        mesh=mesh,
        scratch_types=[
            pltpu.VMEM((NBUF, GROUP), jnp.int32),
            pltpu.VMEM((NBUF, GROUP, D), jnp.float32),
            pltpu.VMEM((NBUF, GROUP, D + 1), jnp.float32),
            pltpu.VMEM((NBUF, GROUP * D), jnp.float32),
            pltpu.VMEM_SHARED((16, GROUP, D), jnp.float32),
        ] + [pltpu.SemaphoreType.DMA] * (2 * NBUF),
        compiler_params=pltpu.CompilerParams(
            use_tc_tiling_on_sc=False, needs_layout_passes=False
        ),
    )
    def emb(table_hbm, idx_hbm, out_hbm, idx_v, rows_v, rpad_v, tbuf_v, shared_v, *sems):
        gsem, wsem = sems[:NBUF], sems[NBUF:]
        wid = lax.axis_index("s") * NC + lax.axis_index("c")
        g0 = wid * g_per_w
        iota = lax.iota(jnp.int32, 16)

        def fetch(g, bf):
            pltpu.sync_copy(idx_hbm.at[pl.ds(g * GROUP, GROUP)], idx_v.at[bf])
            pltpu.async_copy(
                table_hbm.at[idx_v.at[bf]], rows_v.at[bf], gsem[bf]
            )

        def gather_wait(bf):
            pltpu.make_async_copy(
                table_hbm.at[idx_v.at[bf]], rows_v.at[bf], gsem[bf]
            ).wait()
            sid = lax.axis_index("s")
            pltpu.sync_copy(rows_v.at[bf], shared_v.at[sid])
            pltpu.sync_copy(shared_v.at[sid], rpad_v.at[bf, :, pl.ds(0, D)])

        def write_descs(g, bf):
            b = g // groups_per_b
            k0 = (g % groups_per_b) * kp
            descs = []
            for i in range(n_slab):
                off = ((b * n_slab + i) * n_chunk + k0) * 1024
                descs.append(
                    pltpu.make_async_copy(
                        tbuf_v.at[bf, pl.ds(i * kp * 1024, kp * 1024)],
                        out_hbm.at[pl.ds(off, kp * 1024)],
                        wsem[bf],
                    )
                )
            return descs

        def transpose(bf):
            @pl.loop(0, n_t, unroll=8)
            def tstep(t):
                i = t // (kp * 8)
                rem = t - i * (kp * 8)
                k = rem // 8
                s = rem - k * 8
                col = i * 8 + s
                base = k * 128 + iota
                c_idx = jnp.broadcast_to(col, (16,)).astype(jnp.int32)
                for lb in range(8):
                    v = plsc.load_gather(
                        rpad_v.at[bf], [base + lb * 16, c_idx]
                    )
                    tbuf_v[bf, pl.ds(t * 128 + lb * 16, 16)] = v

        for bf in range(NBUF):
            fetch(g0 + bf, bf)

        for bf in range(NBUF):
            g = g0 + bf
            gather_wait(bf)
            transpose(bf)
            for d in write_descs(g, bf):
                d.start()
            if NBUF < g_per_w:
                fetch(g + NBUF, bf)

        @pl.loop(NBUF, g_per_w)
        def grp(gr):
            g = g0 + gr
            bf = lax.rem(gr, NBUF)

            def body(bf_static):
                gather_wait(bf_static)
                for d in write_descs(g - NBUF, bf_static):
                    d.wait()
                transpose(bf_static)
                for d in write_descs(g, bf_static):
                    d.start()

                @pl.when(gr + NBUF < g_per_w)
                def _():
                    fetch(g + NBUF, bf_static)

            for bf_static in range(NBUF):

                @pl.when(bf == bf_static)
                def _():
                    body(bf_static)

        for bf in range(NBUF):
            g = g0 + g_per_w - NBUF + bf
            for d in write_descs(g, bf):
                d.wait()

    return emb


def kernel(x, table):
    A, Bdim = x.shape
    D = table.shape[1]
    xf = jnp.swapaxes(x, 0, 1).reshape(-1).astype(jnp.int32)
    tbl_lin = lax.optimization_barrier(
        table.reshape(table.shape[0] // 2, 2 * D)
    )
    tbl = tbl_lin.reshape(table.shape)
    out_flat = _build(A, Bdim, D)(tbl, xf)
    out5 = out_flat.reshape(Bdim, D // 8, A // 128, 8, 128)
    return out5.transpose(2, 4, 0, 1, 3).reshape(A, Bdim, D)


# speedup vs baseline: 1.5641x; 1.1261x over previous
import functools

import jax
import jax.numpy as jnp
from jax import lax
from jax.experimental import pallas as pl
from jax.experimental.pallas import tpu as pltpu, tpu_sc as plsc

GATHER = 256
NBUF = 5


@functools.lru_cache(maxsize=None)
def _build(B: int, D: int):
    info = plsc.get_sparse_core_info()
    NC, NS = info.num_cores, info.num_subcores
    NW = NC * NS
    assert B % (NW * GATHER * NBUF) == 0
    b_per_w = B // NW
    n_steps = b_per_w // GATHER
    n_groups = n_steps // NBUF

    mesh = plsc.VectorSubcoreMesh(core_axis_name="c", subcore_axis_name="s")

    @functools.partial(
        pl.kernel,
        out_type=jax.ShapeDtypeStruct((B, D), jnp.float32),
        mesh=mesh,
        scratch_types=[
            pltpu.VMEM((b_per_w,), jnp.int32),
            pltpu.VMEM((NBUF, GATHER, D), jnp.float32),
        ] + [pltpu.SemaphoreType.DMA] * (2 * NBUF),
        compiler_params=pltpu.CompilerParams(
            use_tc_tiling_on_sc=False, needs_layout_passes=False
        ),
    )
    def emb(table_hbm, idx_hbm, out_hbm, idx_v, rows_v, *sems):
        gsem, wsem = sems[:NBUF], sems[NBUF:]
        wid = lax.axis_index("s") * NC + lax.axis_index("c")
        base = wid * b_per_w
        pltpu.sync_copy(idx_hbm.at[pl.ds(base, b_per_w)], idx_v)

        def gather_desc(step, b, sem):
            return pltpu.make_async_copy(
                table_hbm.at[idx_v.at[pl.ds(step * GATHER, GATHER)]],
                rows_v.at[b],
                sem,
            )

        def write_desc(step, b, sem):
            return pltpu.make_async_copy(
                rows_v.at[b],
                out_hbm.at[pl.ds(base + step * GATHER, GATHER)],
                sem,
            )

        for b in range(NBUF):
            gather_desc(b, b, gsem[b]).start()

        @pl.loop(0, n_groups - 1)
        def grp(k):
            for b in range(NBUF):
                i = k * NBUF + b
                gather_desc(i, b, gsem[b]).wait()
                write_desc(i, b, wsem[b]).start()
                write_desc(i, b, wsem[b]).wait()
                gather_desc(i + NBUF, b, gsem[b]).start()

        for b in range(NBUF):
            i = (n_groups - 1) * NBUF + b
            gather_desc(i, b, gsem[b]).wait()
            write_desc(i, b, wsem[b]).start()
        for b in range(NBUF):
            i = (n_groups - 1) * NBUF + b
            write_desc(i, b, wsem[b]).wait()

    return emb


def kernel(x, table):
    A, Bdim = x.shape
    D = table.shape[1]
    xf = jnp.swapaxes(x, 0, 1).reshape(-1).astype(jnp.int32)
    tbl_lin = lax.optimization_barrier(
        table.reshape(table.shape[0] // 2, 2 * D)
    )
    tbl = tbl_lin.reshape(table.shape)
    out = _build(xf.shape[0], D)(tbl, xf)
    return out.reshape(Bdim, A, D).transpose(1, 0, 2)
